# baseline (speedup 1.0000x reference)
import functools

import jax
import jax.numpy as jnp
from jax import lax
from jax.experimental import pallas as pl
from jax.experimental.pallas import tpu as pltpu
from jax.experimental.pallas import tpu_sc as plsc

NTOK = 4 * 8192
D = 1024
NC, NS = 2, 16
NW = NC * NS
HALF = NTOK // 2
PER_W = HALF // NW
CH = 1024
NCH = NTOK // CH
K = 4


@functools.lru_cache(maxsize=None)
def _make_flag_gather(vocab):
    mesh = plsc.VectorSubcoreMesh(core_axis_name="c", subcore_axis_name="s")

    @functools.partial(
        pl.kernel,
        mesh=mesh,
        out_type=jax.ShapeDtypeStruct((HALF,), jnp.int32),
        scratch_types=[
            pltpu.VMEM((PER_W,), jnp.int32),
            pltpu.VMEM((PER_W,), jnp.int32),
            pltpu.SemaphoreType.DMA,
        ],
    )
    def gather_flags(ids_hbm, table_hbm, out_hbm, idx_v, flags_v, sem):
        wid = lax.axis_index("s") * NC + lax.axis_index("c")
        base = wid * PER_W
        pltpu.sync_copy(ids_hbm.at[pl.ds(base, PER_W)], idx_v)
        for i in range(PER_W // 16):
            sl = pl.ds(i * 16, 16)
            idx_v[sl] = jnp.minimum(idx_v[sl], vocab)
        pltpu.async_copy(table_hbm.at[idx_v], flags_v, sem).wait()
        pltpu.sync_copy(flags_v, out_hbm.at[pl.ds(base, PER_W)])

    return gather_flags


def _make_tc_body(g0, g1):
    def tc_body(f_hbm, w_ref, e_hbm, o_hbm, ebufs, obufs, fbufs,
                esems, fsems, osems):
        def start_read(g, slot):
            pltpu.make_async_copy(
                e_hbm.at[pl.ds(g * CH, CH), :], ebufs.at[slot], esems.at[slot]
            ).start()
            pltpu.make_async_copy(
                f_hbm.at[pl.ds((g - g0) * CH, CH)], fbufs.at[slot],
                fsems.at[slot]
            ).start()

        def out_copy(g, slot):
            return pltpu.make_async_copy(
                obufs.at[slot], o_hbm.at[pl.ds(g * CH, CH), :], osems.at[slot]
            )

        for i, g in enumerate(range(g0, min(g0 + K, g1))):
            start_read(g, i)

        w0 = w_ref[0:1, :]
        dw = w_ref[1:2, :] - w0

        for g in range(g0, g1):
            slot = (g - g0) % K
            pltpu.make_async_copy(
                e_hbm.at[pl.ds(g * CH, CH), :], ebufs.at[slot], esems.at[slot]
            ).wait()
            pltpu.make_async_copy(
                f_hbm.at[pl.ds((g - g0) * CH, CH)], fbufs.at[slot],
                fsems.at[slot]
            ).wait()
            if g - g0 >= K:
                out_copy(g - K, slot).wait()
            f = fbufs[slot].astype(jnp.float32).reshape(CH, 1)
            obufs[slot] = ebufs[slot] + (w0 + f * dw)
            out_copy(g, slot).start()
            nxt = g + K
            if nxt < g1:
                start_read(nxt, slot)

        for g in range(max(g0, g1 - K), g1):
            out_copy(g, (g - g0) % K).wait()

    return tc_body


_SCRATCH = [
    pltpu.VMEM((K, CH, D), jnp.float32),
    pltpu.VMEM((K, CH, D), jnp.float32),
    pltpu.VMEM((K, CH), jnp.int32),
    pltpu.SemaphoreType.DMA((K,)),
    pltpu.SemaphoreType.DMA((K,)),
    pltpu.SemaphoreType.DMA((K,)),
]


def kernel(subword_embeds, token_ids, is_continuation, cont_emb_weight):
    vocab = is_continuation.shape[0] - 1
    ids = token_ids.astype(jnp.int32).reshape(NTOK)
    table = is_continuation.astype(jnp.int32)
    w = cont_emb_weight.astype(jnp.float32)
    e2d = subword_embeds.reshape(NTOK, D)

    gather = _make_flag_gather(vocab)
    flags_a = gather(ids[:HALF], table)
    flags_b = gather(ids[HALF:], table)

    half_chunks = NCH // 2
    out1 = pl.pallas_call(
        _make_tc_body(0, half_chunks),
        in_specs=[
            pl.BlockSpec(memory_space=pl.ANY),
            pl.BlockSpec((2, D), lambda: (0, 0)),
            pl.BlockSpec(memory_space=pl.ANY),
        ],
        out_specs=pl.BlockSpec(memory_space=pl.ANY),
        out_shape=jax.ShapeDtypeStruct((NTOK, D), jnp.float32),
        scratch_shapes=_SCRATCH,
    )(flags_a, w, e2d)

    def tc2_body(f_hbm, w_ref, e_hbm, prev_hbm, o_hbm, *scratch):
        _make_tc_body(half_chunks, NCH)(f_hbm, w_ref, e_hbm, o_hbm, *scratch)

    out = pl.pallas_call(
        tc2_body,
        in_specs=[
            pl.BlockSpec(memory_space=pl.ANY),
            pl.BlockSpec((2, D), lambda: (0, 0)),
            pl.BlockSpec(memory_space=pl.ANY),
            pl.BlockSpec(memory_space=pl.ANY),
        ],
        out_specs=pl.BlockSpec(memory_space=pl.ANY),
        out_shape=jax.ShapeDtypeStruct((NTOK, D), jnp.float32),
        scratch_shapes=_SCRATCH,
        input_output_aliases={3: 0},
    )(flags_b, w, e2d, out1)
    return out.reshape(subword_embeds.shape)


# speedup vs baseline: 1.0081x; 1.0081x over previous
import functools

import jax
import jax.numpy as jnp
from jax import lax
from jax.experimental import pallas as pl
from jax.experimental.pallas import tpu as pltpu
from jax.experimental.pallas import tpu_sc as plsc

NTOK = 4 * 8192
D = 1024
NC, NS = 2, 16
NW = NC * NS
PER_W = NTOK // NW
CH = 1024
NCH = NTOK // CH
K = 6


@functools.lru_cache(maxsize=None)
def _make_flag_gather(vocab):
    mesh = plsc.VectorSubcoreMesh(core_axis_name="c", subcore_axis_name="s")

    @functools.partial(
        pl.kernel,
        mesh=mesh,
        out_type=jax.ShapeDtypeStruct((NTOK,), jnp.int32),
        scratch_types=[
            pltpu.VMEM((PER_W,), jnp.int32),
            pltpu.VMEM((PER_W,), jnp.int32),
            pltpu.SemaphoreType.DMA,
        ],
    )
    def gather_flags(ids_hbm, table_hbm, out_hbm, idx_v, flags_v, sem):
        wid = lax.axis_index("s") * NC + lax.axis_index("c")
        base = wid * PER_W
        pltpu.sync_copy(ids_hbm.at[pl.ds(base, PER_W)], idx_v)
        for i in range(PER_W // 16):
            sl = pl.ds(i * 16, 16)
            idx_v[sl] = jnp.minimum(idx_v[sl], vocab)
        pltpu.async_copy(table_hbm.at[idx_v], flags_v, sem).wait()
        pltpu.sync_copy(flags_v, out_hbm.at[pl.ds(base, PER_W)])

    return gather_flags


def _tc_body(f_hbm, w_ref, e_hbm, o_hbm, ebufs, obufs, fbufs,
             esems, fsems, osems):
    def start_read(g, slot):
        pltpu.make_async_copy(
            e_hbm.at[pl.ds(g * CH, CH), :], ebufs.at[slot], esems.at[slot]
        ).start()
        pltpu.make_async_copy(
            f_hbm.at[pl.ds(g * CH, CH)], fbufs.at[slot], fsems.at[slot]
        ).start()

    def out_copy(g, slot):
        return pltpu.make_async_copy(
            obufs.at[slot], o_hbm.at[pl.ds(g * CH, CH), :], osems.at[slot]
        )

    for slot in range(K):
        start_read(slot, slot)

    w0 = w_ref[0:1, :]
    dw = w_ref[1:2, :] - w0

    for g in range(NCH):
        slot = g % K
        pltpu.make_async_copy(
            e_hbm.at[pl.ds(g * CH, CH), :], ebufs.at[slot], esems.at[slot]
        ).wait()
        pltpu.make_async_copy(
            f_hbm.at[pl.ds(g * CH, CH)], fbufs.at[slot], fsems.at[slot]
        ).wait()
        if g >= K:
            out_copy(g - K, slot).wait()
        f = fbufs[slot].astype(jnp.float32).reshape(CH, 1)
        obufs[slot] = ebufs[slot] + (w0 + f * dw)
        out_copy(g, slot).start()
        nxt = g + K
        if nxt < NCH:
            start_read(nxt, slot)

    for g in range(NCH - K, NCH):
        out_copy(g, g % K).wait()


def kernel(subword_embeds, token_ids, is_continuation, cont_emb_weight):
    vocab = is_continuation.shape[0] - 1
    ids = token_ids.astype(jnp.int32).reshape(NTOK)
    table = is_continuation.astype(jnp.int32)

    flags = _make_flag_gather(vocab)(ids, table)

    e2d = subword_embeds.reshape(NTOK, D)
    out = pl.pallas_call(
        _tc_body,
        in_specs=[
            pl.BlockSpec(memory_space=pl.ANY),
            pl.BlockSpec((2, D), lambda: (0, 0)),
            pl.BlockSpec(memory_space=pl.ANY),
        ],
        out_specs=pl.BlockSpec(memory_space=pl.ANY),
        out_shape=jax.ShapeDtypeStruct((NTOK, D), jnp.float32),
        scratch_shapes=[
            pltpu.VMEM((K, CH, D), jnp.float32),
            pltpu.VMEM((K, CH, D), jnp.float32),
            pltpu.VMEM((K, CH), jnp.int32),
            pltpu.SemaphoreType.DMA((K,)),
            pltpu.SemaphoreType.DMA((K,)),
            pltpu.SemaphoreType.DMA((K,)),
        ],
    )(flags, cont_emb_weight.astype(jnp.float32), e2d)
    return out.reshape(subword_embeds.shape)


# speedup vs baseline: 1.0130x; 1.0049x over previous
import functools

import jax
import jax.numpy as jnp
from jax import lax
from jax.experimental import pallas as pl
from jax.experimental.pallas import tpu as pltpu
from jax.experimental.pallas import tpu_sc as plsc

NTOK = 4 * 8192
D = 1024
NC, NS = 2, 16
NW = NC * NS
PER_W = NTOK // NW
CH = 1024
NCH = NTOK // CH
K = 4


@functools.lru_cache(maxsize=None)
def _make_flag_gather(vocab):
    mesh = plsc.VectorSubcoreMesh(core_axis_name="c", subcore_axis_name="s")

    @functools.partial(
        pl.kernel,
        mesh=mesh,
        out_type=jax.ShapeDtypeStruct((NTOK,), jnp.int32),
        scratch_types=[
            pltpu.VMEM((PER_W,), jnp.int32),
            pltpu.VMEM((PER_W,), jnp.int32),
            pltpu.SemaphoreType.DMA((6,)),
        ],
    )
    def gather_flags(ids_hbm, table_hbm, out_hbm, idx_v, flags_v, sems):
        wid = lax.axis_index("s") * NC + lax.axis_index("c")
        base = wid * PER_W
        H2 = PER_W // 2
        ld = [pltpu.make_async_copy(
                  ids_hbm.at[pl.ds(base + h * H2, H2)],
                  idx_v.at[pl.ds(h * H2, H2)], sems.at[h])
              for h in range(2)]
        gt = [pltpu.make_async_copy(
                  table_hbm.at[idx_v.at[pl.ds(h * H2, H2)]],
                  flags_v.at[pl.ds(h * H2, H2)], sems.at[2 + h])
              for h in range(2)]
        st = [pltpu.make_async_copy(
                  flags_v.at[pl.ds(h * H2, H2)],
                  out_hbm.at[pl.ds(base + h * H2, H2)], sems.at[4 + h])
              for h in range(2)]
        ld[0].start()
        ld[1].start()
        for h in range(2):
            ld[h].wait()
            for i in range(h * H2 // 16, (h + 1) * H2 // 16):
                sl = pl.ds(i * 16, 16)
                idx_v[sl] = jnp.minimum(idx_v[sl], vocab)
            gt[h].start()
        for h in range(2):
            gt[h].wait()
            st[h].start()
        st[0].wait()
        st[1].wait()

    return gather_flags


def _tc_body(f_hbm, w_ref, e_hbm, o_hbm, ebufs, obufs, fbufs,
             esems, fsems, osems):
    def start_read(g, slot):
        pltpu.make_async_copy(
            e_hbm.at[pl.ds(g * CH, CH), :], ebufs.at[slot], esems.at[slot]
        ).start()
        pltpu.make_async_copy(
            f_hbm.at[pl.ds(g * CH, CH)], fbufs.at[slot], fsems.at[slot]
        ).start()

    def out_copy(g, slot):
        return pltpu.make_async_copy(
            obufs.at[slot], o_hbm.at[pl.ds(g * CH, CH), :], osems.at[slot]
        )

    for slot in range(K):
        start_read(slot, slot)

    w0 = w_ref[0:1, :]
    dw = w_ref[1:2, :] - w0

    for g in range(NCH):
        slot = g % K
        pltpu.make_async_copy(
            e_hbm.at[pl.ds(g * CH, CH), :], ebufs.at[slot], esems.at[slot]
        ).wait()
        pltpu.make_async_copy(
            f_hbm.at[pl.ds(g * CH, CH)], fbufs.at[slot], fsems.at[slot]
        ).wait()
        if g >= K:
            out_copy(g - K, slot).wait()
        f = fbufs[slot].astype(jnp.float32).reshape(CH, 1)
        obufs[slot] = ebufs[slot] + (w0 + f * dw)
        out_copy(g, slot).start()
        nxt = g + K
        if nxt < NCH:
            start_read(nxt, slot)

    for g in range(NCH - K, NCH):
        out_copy(g, g % K).wait()


def kernel(subword_embeds, token_ids, is_continuation, cont_emb_weight):
    vocab = is_continuation.shape[0] - 1
    ids = token_ids.astype(jnp.int32).reshape(NTOK)
    table = is_continuation.astype(jnp.int32)

    flags = _make_flag_gather(vocab)(ids, table)

    e2d = subword_embeds.reshape(NTOK, D)
    out = pl.pallas_call(
        _tc_body,
        in_specs=[
            pl.BlockSpec(memory_space=pl.ANY),
            pl.BlockSpec((2, D), lambda: (0, 0)),
            pl.BlockSpec(memory_space=pl.ANY),
        ],
        out_specs=pl.BlockSpec(memory_space=pl.ANY),
        out_shape=jax.ShapeDtypeStruct((NTOK, D), jnp.float32),
        scratch_shapes=[
            pltpu.VMEM((K, CH, D), jnp.float32),
            pltpu.VMEM((K, CH, D), jnp.float32),
            pltpu.VMEM((K, CH), jnp.int32),
            pltpu.SemaphoreType.DMA((K,)),
            pltpu.SemaphoreType.DMA((K,)),
            pltpu.SemaphoreType.DMA((K,)),
        ],
    )(flags, cont_emb_weight.astype(jnp.float32), e2d)
    return out.reshape(subword_embeds.shape)


# speedup vs baseline: 1.0131x; 1.0000x over previous
import functools

import jax
import jax.numpy as jnp
from jax import lax
from jax.experimental import pallas as pl
from jax.experimental.pallas import tpu as pltpu
from jax.experimental.pallas import tpu_sc as plsc

NTOK = 4 * 8192
D = 1024
NC, NS = 2, 16
NW = NC * NS
PER_W = NTOK // NW
CH = 1024
NCH = NTOK // CH
K = 4


@functools.lru_cache(maxsize=None)
def _make_flag_gather(vocab):
    mesh = plsc.VectorSubcoreMesh(core_axis_name="c", subcore_axis_name="s")

    @functools.partial(
        pl.kernel,
        mesh=mesh,
        out_type=jax.ShapeDtypeStruct((NTOK,), jnp.int32),
        scratch_types=[
            pltpu.VMEM((PER_W,), jnp.int32),
            pltpu.VMEM((PER_W,), jnp.int32),
            pltpu.SemaphoreType.DMA((6,)),
        ],
    )
    def gather_flags(ids_hbm, table_hbm, out_hbm, idx_v, flags_v, sems):
        wid = lax.axis_index("s") * NC + lax.axis_index("c")
        base = wid * PER_W
        row = wid // 8
        col = (wid % 8) * PER_W
        H2 = PER_W // 2
        ld = [pltpu.make_async_copy(
                  ids_hbm.at[row, pl.ds(col + h * H2, H2)],
                  idx_v.at[pl.ds(h * H2, H2)], sems.at[h])
              for h in range(2)]
        gt = [pltpu.make_async_copy(
                  table_hbm.at[idx_v.at[pl.ds(h * H2, H2)]],
                  flags_v.at[pl.ds(h * H2, H2)], sems.at[2 + h])
              for h in range(2)]
        st = [pltpu.make_async_copy(
                  flags_v.at[pl.ds(h * H2, H2)],
                  out_hbm.at[pl.ds(base + h * H2, H2)], sems.at[4 + h])
              for h in range(2)]
        ld[0].start()
        ld[1].start()
        for h in range(2):
            ld[h].wait()
            for i in range(h * H2 // 16, (h + 1) * H2 // 16):
                sl = pl.ds(i * 16, 16)
                idx_v[sl] = jnp.minimum(idx_v[sl], vocab)
            gt[h].start()
        for h in range(2):
            gt[h].wait()
            st[h].start()
        st[0].wait()
        st[1].wait()

    return gather_flags


def _tc_body(f_hbm, w_ref, e_hbm, o_hbm, ebufs, obufs, fbufs,
             esems, fsems, osems):
    def start_read(g, slot):
        pltpu.make_async_copy(
            e_hbm.at[pl.ds(g * CH, CH), :], ebufs.at[slot], esems.at[slot]
        ).start()
        pltpu.make_async_copy(
            f_hbm.at[pl.ds(g * CH, CH)], fbufs.at[slot], fsems.at[slot]
        ).start()

    def out_copy(g, slot):
        return pltpu.make_async_copy(
            obufs.at[slot], o_hbm.at[pl.ds(g * CH, CH), :], osems.at[slot]
        )

    for slot in range(K):
        start_read(slot, slot)

    w0 = w_ref[0:1, :]
    dw = w_ref[1:2, :] - w0

    for g in range(NCH):
        slot = g % K
        pltpu.make_async_copy(
            e_hbm.at[pl.ds(g * CH, CH), :], ebufs.at[slot], esems.at[slot]
        ).wait()
        pltpu.make_async_copy(
            f_hbm.at[pl.ds(g * CH, CH)], fbufs.at[slot], fsems.at[slot]
        ).wait()
        if g >= K:
            out_copy(g - K, slot).wait()
        f = fbufs[slot].astype(jnp.float32).reshape(CH, 1)
        obufs[slot] = ebufs[slot] + (w0 + f * dw)
        out_copy(g, slot).start()
        nxt = g + K
        if nxt < NCH:
            start_read(nxt, slot)

    for g in range(NCH - K, NCH):
        out_copy(g, g % K).wait()


def kernel(subword_embeds, token_ids, is_continuation, cont_emb_weight):
    vocab = is_continuation.shape[0] - 1
    ids = token_ids.astype(jnp.int32)
    table = is_continuation.astype(jnp.int32)

    flags = _make_flag_gather(vocab)(ids, table)

    e2d = subword_embeds.reshape(NTOK, D)
    out = pl.pallas_call(
        _tc_body,
        in_specs=[
            pl.BlockSpec(memory_space=pl.ANY),
            pl.BlockSpec((2, D), lambda: (0, 0)),
            pl.BlockSpec(memory_space=pl.ANY),
        ],
        out_specs=pl.BlockSpec(memory_space=pl.ANY),
        out_shape=jax.ShapeDtypeStruct((NTOK, D), jnp.float32),
        scratch_shapes=[
            pltpu.VMEM((K, CH, D), jnp.float32),
            pltpu.VMEM((K, CH, D), jnp.float32),
            pltpu.VMEM((K, CH), jnp.int32),
            pltpu.SemaphoreType.DMA((K,)),
            pltpu.SemaphoreType.DMA((K,)),
            pltpu.SemaphoreType.DMA((K,)),
        ],
    )(flags, cont_emb_weight.astype(jnp.float32), e2d)
    return out.reshape(subword_embeds.shape)
